# baseline (speedup 1.0000x reference)
import functools

import jax
import jax.numpy as jnp
from jax import lax
from jax.experimental import pallas as pl
from jax.experimental.pallas import tpu as pltpu
from jax.experimental.pallas import tpu_sc as plsc

C = 64
NCHUNK = 4
EB = 80
INV_SQRT = 1.0 / (1.0 + 1e-5) ** 0.5


def _proj_nodes_body(x_ref, w_ref, b_ref, ax_ref, bt_ref, dt_ref, et_ref):
    y = jnp.dot(x_ref[...], w_ref[...], preferred_element_type=jnp.float32)
    y = y + b_ref[...]
    d = ax_ref.shape[1]
    ax_ref[...] = y[:, :d]
    for c in range(NCHUNK):
        bt_ref[c] = y[:, d + c * C:d + (c + 1) * C]
        dt_ref[c] = y[:, 2 * d + c * C:2 * d + (c + 1) * C]
        et_ref[c] = y[:, 3 * d + c * C:3 * d + (c + 1) * C]


def _proj_nodes(x, w, b):
    n, d = x.shape
    bn = 2000
    grid = (n // bn,)
    return pl.pallas_call(
        _proj_nodes_body,
        grid=grid,
        in_specs=[
            pl.BlockSpec((bn, d), lambda i: (i, 0)),
            pl.BlockSpec((d, 4 * d), lambda i: (0, 0)),
            pl.BlockSpec((1, 4 * d), lambda i: (0, 0)),
        ],
        out_specs=[
            pl.BlockSpec((bn, d), lambda i: (i, 0)),
            pl.BlockSpec((NCHUNK, bn, C), lambda i: (0, i, 0)),
            pl.BlockSpec((NCHUNK, bn, C), lambda i: (0, i, 0)),
            pl.BlockSpec((NCHUNK, bn, C), lambda i: (0, i, 0)),
        ],
        out_shape=[
            jax.ShapeDtypeStruct((n, d), jnp.float32),
            jax.ShapeDtypeStruct((NCHUNK, n, C), jnp.float32),
            jax.ShapeDtypeStruct((NCHUNK, n, C), jnp.float32),
            jax.ShapeDtypeStruct((NCHUNK, n, C), jnp.float32),
        ],
    )(x, w, b)


def _proj_edges_body(ea_ref, w_ref, b_ref, ct_ref):
    y = jnp.dot(ea_ref[...], w_ref[...], preferred_element_type=jnp.float32)
    y = y + b_ref[...]
    for c in range(NCHUNK):
        ct_ref[c] = y[:, c * C:(c + 1) * C]


def _proj_edges(ea, w, b):
    e, d = ea.shape
    be = 2000
    grid = (e // be,)
    return pl.pallas_call(
        _proj_edges_body,
        grid=grid,
        in_specs=[
            pl.BlockSpec((be, d), lambda i: (i, 0)),
            pl.BlockSpec((d, d), lambda i: (0, 0)),
            pl.BlockSpec((1, d), lambda i: (0, 0)),
        ],
        out_specs=[pl.BlockSpec((NCHUNK, be, C), lambda i: (0, i, 0))],
        out_shape=[jax.ShapeDtypeStruct((NCHUNK, e, C), jnp.float32)],
    )(ea, w, b)[0]


def _sc_edge_body(dst_hbm, src_hbm, dtab, etab, btab, ctab, zeros_hbm,
                  ech_hbm, agg_hbm,
                  dstb, srcb, dsto, srco, dxb, exb, bxb, ceb, eob, msb,
                  acc, sem0, sem1, sem2, sem3):
    n = dtab.shape[0] // NCHUNK
    e = dst_hbm.shape[0]
    npad = agg_hbm.shape[1]
    cid = lax.axis_index("c")
    sid = lax.axis_index("s")
    ept = e // 16
    nblk = ept // EB
    npt = npad // 16
    tbase = sid * ept

    for r in range(2):
        chunk = cid * 2 + r
        pltpu.sync_copy(zeros_hbm, acc.at[pl.ds(sid * npt, npt), :])
        plsc.subcore_barrier()

        def blk(i, _):
            e0 = tbase + i * EB
            pltpu.sync_copy(dst_hbm.at[pl.ds(e0, EB)], dstb)
            pltpu.sync_copy(src_hbm.at[pl.ds(e0, EB)], srcb)
            off = chunk * n
            for k in range(EB // 16):
                s = pl.ds(k * 16, 16)
                dsto[s] = dstb[s] + off
                srco[s] = srcb[s] + off
            cp0 = pltpu.async_copy(dtab.at[dsto], dxb, sem0)
            cp1 = pltpu.async_copy(etab.at[srco], exb, sem1)
            cp2 = pltpu.async_copy(btab.at[srco], bxb, sem2)
            cp3 = pltpu.async_copy(ctab.at[pl.ds(chunk * e + e0, EB), :], ceb, sem3)
            cp0.wait()
            cp1.wait()
            cp2.wait()
            cp3.wait()

            def row_body(row, _):
                for j in range(C // 16):
                    s = pl.ds(j * 16, 16)
                    ev = dxb[row, s] + exb[row, s] + ceb[row, s]
                    sig = 1.0 / (1.0 + jnp.exp(-ev))
                    msb[row, s] = sig * bxb[row, s]
                    msb[row, pl.ds(C + j * 16, 16)] = sig
                    eob[row, s] = ev
                return 0

            lax.fori_loop(0, EB, row_body, 0)
            cpo = pltpu.async_copy(
                eob, ech_hbm.at[pl.ds(chunk * e + e0, EB), :], sem3)
            pltpu.sync_copy(msb, acc.at[dstb], add=True)
            cpo.wait()
            return 0

        lax.fori_loop(0, nblk, blk, 0)
        plsc.subcore_barrier()
        pltpu.sync_copy(acc.at[pl.ds(sid * npt, npt), :],
                        agg_hbm.at[chunk, pl.ds(sid * npt, npt), :])
        plsc.subcore_barrier()


def _sc_edge(dst, src, dtab, etab, btab, ctab, zeros, e, npad):
    mesh = plsc.VectorSubcoreMesh(core_axis_name="c", subcore_axis_name="s")
    kern = pl.kernel(
        _sc_edge_body,
        out_type=(
            jax.ShapeDtypeStruct((NCHUNK * e, C), jnp.float32),
            jax.ShapeDtypeStruct((NCHUNK, npad, 2 * C), jnp.float32),
        ),
        mesh=mesh,
        compiler_params=pltpu.CompilerParams(use_tc_tiling_on_sc=False),
        scratch_types=[
            pltpu.VMEM((EB,), jnp.int32),
            pltpu.VMEM((EB,), jnp.int32),
            pltpu.VMEM((EB,), jnp.int32),
            pltpu.VMEM((EB,), jnp.int32),
            pltpu.VMEM((EB, C), jnp.float32),
            pltpu.VMEM((EB, C), jnp.float32),
            pltpu.VMEM((EB, C), jnp.float32),
            pltpu.VMEM((EB, C), jnp.float32),
            pltpu.VMEM((EB, C), jnp.float32),
            pltpu.VMEM((EB, 2 * C), jnp.float32),
            pltpu.VMEM_SHARED((npad, 2 * C), jnp.float32),
            pltpu.SemaphoreType.DMA,
            pltpu.SemaphoreType.DMA,
            pltpu.SemaphoreType.DMA,
            pltpu.SemaphoreType.DMA,
        ],
    )
    return kern(dst, src, dtab, etab, btab, ctab, zeros)


def _combine_body(x_ref, ax_ref, agg_ref, out_ref):
    for c in range(NCHUNK):
        s = pl.ds(c * C, C)
        num = agg_ref[c, :, :C]
        den = agg_ref[c, :, C:]
        t = (ax_ref[:, s] + num / (den + 1e-6)) * INV_SQRT
        out_ref[:, s] = x_ref[:, s] + jnp.maximum(t, 0.0)


def _combine(x, ax, agg):
    n, d = x.shape
    bn = 2000
    grid = (n // bn,)
    return pl.pallas_call(
        _combine_body,
        grid=grid,
        in_specs=[
            pl.BlockSpec((bn, d), lambda i: (i, 0)),
            pl.BlockSpec((bn, d), lambda i: (i, 0)),
            pl.BlockSpec((NCHUNK, bn, 2 * C), lambda i: (0, i, 0)),
        ],
        out_specs=pl.BlockSpec((bn, d), lambda i: (i, 0)),
        out_shape=jax.ShapeDtypeStruct((n, d), jnp.float32),
    )(x, ax, agg)


def _eout_body(ea_ref, ech_ref, out_ref):
    for c in range(NCHUNK):
        s = pl.ds(c * C, C)
        out_ref[:, s] = ea_ref[:, s] + jnp.maximum(ech_ref[c] * INV_SQRT, 0.0)


def _eout(ea, ech):
    e, d = ea.shape
    be = 2000
    grid = (e // be,)
    return pl.pallas_call(
        _eout_body,
        grid=grid,
        in_specs=[
            pl.BlockSpec((be, d), lambda i: (i, 0)),
            pl.BlockSpec((NCHUNK, be, C), lambda i: (0, i, 0)),
        ],
        out_specs=pl.BlockSpec((be, d), lambda i: (i, 0)),
        out_shape=jax.ShapeDtypeStruct((e, d), jnp.float32),
    )(ea, ech)


@jax.jit
def kernel(x, edge_index, edge_attr, W_A, b_A, W_B, b_B, W_C, b_C,
           W_D, b_D, W_E, b_E):
    n, d = x.shape
    e = edge_attr.shape[0]
    src = edge_index[0]
    dst = edge_index[1]

    wn = jnp.concatenate([W_A, W_B, W_D, W_E], axis=1)
    bn = jnp.concatenate([b_A, b_B, b_D, b_E]).reshape(1, 4 * d)
    ax, btab, dtab, etab = _proj_nodes(x, wn, bn)
    ctab = _proj_edges(edge_attr, W_C, b_C.reshape(1, d))

    npad = 10240
    zeros = jnp.zeros((npad // 16, 2 * C), jnp.float32)
    ech, agg = _sc_edge(
        dst, src,
        dtab.reshape(NCHUNK * n, C),
        etab.reshape(NCHUNK * n, C),
        btab.reshape(NCHUNK * n, C),
        ctab.reshape(NCHUNK * e, C),
        zeros, e, npad)

    xout = _combine(x, ax, agg)
    eout = _eout(edge_attr, ech.reshape(NCHUNK, e, C))
    return (xout, eout)


# speedup vs baseline: 1.6583x; 1.6583x over previous
import functools

import jax
import jax.numpy as jnp
from jax import lax
from jax.experimental import pallas as pl
from jax.experimental.pallas import tpu as pltpu
from jax.experimental.pallas import tpu_sc as plsc

C = 32
NCHUNK = 8
CPC = NCHUNK // 2
EB = 80
NBLK = 125
INV_SQRT = 1.0 / (1.0 + 1e-5) ** 0.5


def _proj_nodes_body(x_ref, w_ref, b_ref, ax_ref, bt_ref, dt_ref, et_ref):
    y = jnp.dot(x_ref[...], w_ref[...], preferred_element_type=jnp.float32)
    y = y + b_ref[...]
    d = ax_ref.shape[1]
    ax_ref[...] = y[:, :d]
    for c in range(NCHUNK):
        bt_ref[c] = y[:, d + c * C:d + (c + 1) * C]
        dt_ref[c] = y[:, 2 * d + c * C:2 * d + (c + 1) * C]
        et_ref[c] = y[:, 3 * d + c * C:3 * d + (c + 1) * C]


def _proj_nodes(x, w, b):
    n, d = x.shape
    bn = 1000
    grid = (n // bn,)
    return pl.pallas_call(
        _proj_nodes_body,
        grid=grid,
        in_specs=[
            pl.BlockSpec((bn, d), lambda i: (i, 0)),
            pl.BlockSpec((d, 4 * d), lambda i: (0, 0)),
            pl.BlockSpec((1, 4 * d), lambda i: (0, 0)),
        ],
        out_specs=[
            pl.BlockSpec((bn, d), lambda i: (i, 0)),
            pl.BlockSpec((NCHUNK, bn, C), lambda i: (0, i, 0)),
            pl.BlockSpec((NCHUNK, bn, C), lambda i: (0, i, 0)),
            pl.BlockSpec((NCHUNK, bn, C), lambda i: (0, i, 0)),
        ],
        out_shape=[
            jax.ShapeDtypeStruct((n, d), jnp.float32),
            jax.ShapeDtypeStruct((NCHUNK, n, C), jnp.float32),
            jax.ShapeDtypeStruct((NCHUNK, n, C), jnp.float32),
            jax.ShapeDtypeStruct((NCHUNK, n, C), jnp.float32),
        ],
    )(x, w, b)


def _proj_edges_body(ea_ref, w_ref, b_ref, ct_ref):
    y = jnp.dot(ea_ref[...], w_ref[...], preferred_element_type=jnp.float32)
    y = y + b_ref[...]
    for c in range(NCHUNK):
        ct_ref[c] = y[:, c * C:(c + 1) * C]


def _proj_edges(ea, w, b):
    e, d = ea.shape
    be = 2000
    grid = (e // be,)
    return pl.pallas_call(
        _proj_edges_body,
        grid=grid,
        in_specs=[
            pl.BlockSpec((be, d), lambda i: (i, 0)),
            pl.BlockSpec((d, d), lambda i: (0, 0)),
            pl.BlockSpec((1, d), lambda i: (0, 0)),
        ],
        out_specs=[pl.BlockSpec((NCHUNK, be, C), lambda i: (0, i, 0))],
        out_shape=[jax.ShapeDtypeStruct((NCHUNK, e, C), jnp.float32)],
    )(ea, w, b)[0]


def _sc_edge_body(dst2d_hbm, src2d_hbm, dtab, etab, btab, ctab,
                  zeros_hbm, ech_hbm, agg_hbm,
                  dst2d, dstoff, srcoff,
                  dx0, ex0, bx0, ce0, eo0, ms0,
                  dx1, ex1, bx1, ce1, eo1, ms1,
                  acc, sg0, sg1, ss0, ss1, se0, se1):
    n = dtab.shape[0] // NCHUNK
    e = NBLK * EB * 16
    npad = agg_hbm.shape[1]
    cid = lax.axis_index("c")
    sid = lax.axis_index("s")
    ept = NBLK * EB
    npt = npad // 16
    tbase = sid * ept

    pltpu.sync_copy(dst2d_hbm.at[sid], dst2d)
    pltpu.sync_copy(dst2d_hbm.at[sid], dstoff)
    pltpu.sync_copy(src2d_hbm.at[sid], srcoff)

    bufs = ((dx0, ex0, bx0, ce0, eo0, ms0, sg0, ss0, se0),
            (dx1, ex1, bx1, ce1, eo1, ms1, sg1, ss1, se1))

    def add_off(ref, off):
        vec = jnp.full((16,), off, dtype=jnp.int32)

        @pl.loop(0, NBLK)
        def _(i):
            for j in range(EB // 16):
                sl = pl.ds(j * 16, 16)
                ref[i, sl] = ref[i, sl] + vec

    def issue_gathers(b, bs, chunk):
        dx, ex, bx, ce, _, _, sg, _, _ = bs
        pltpu.async_copy(dtab.at[dstoff.at[b]], dx, sg)
        pltpu.async_copy(etab.at[srcoff.at[b]], ex, sg)
        pltpu.async_copy(btab.at[srcoff.at[b]], bx, sg)
        pltpu.async_copy(ctab.at[pl.ds(chunk * e + tbase + b * EB, EB), :],
                         ce, sg)

    def wait_gathers(b, bs, chunk):
        dx, ex, bx, ce, _, _, sg, _, _ = bs
        pltpu.make_async_copy(dtab.at[dstoff.at[b]], dx, sg).wait()
        pltpu.make_async_copy(etab.at[srcoff.at[b]], ex, sg).wait()
        pltpu.make_async_copy(btab.at[srcoff.at[b]], bx, sg).wait()
        pltpu.make_async_copy(
            ctab.at[pl.ds(chunk * e + tbase + b * EB, EB), :], ce, sg).wait()

    def compute(b, bs):
        dx, ex, bx, ce, eo, ms, _, _, _ = bs

        @plsc.parallel_loop(0, EB)
        def _(row):
            for j in range(C // 16):
                s = pl.ds(j * 16, 16)
                ev = dx[row, s] + ex[row, s] + ce[row, s]
                eo[row, s] = ev
                sig = 1.0 / (1.0 + jnp.exp(-ev))
                ms[row, s] = sig * bx[row, s]
                ms[row, pl.ds(C + j * 16, 16)] = sig

    def issue_out(b, bs, chunk):
        _, _, _, _, eo, ms, _, ss, se = bs
        pltpu.async_copy(
            eo, ech_hbm.at[pl.ds(chunk * e + tbase + b * EB, EB), :], se)
        pltpu.async_copy(ms, acc.at[dst2d.at[b]], ss, add=True)

    def wait_out(b, bs, chunk):
        _, _, _, _, eo, ms, _, ss, se = bs
        pltpu.make_async_copy(
            eo, ech_hbm.at[pl.ds(chunk * e + tbase + b * EB, EB), :], se).wait()
        pltpu.make_async_copy(ms, acc.at[dst2d.at[b]], ss).wait()

    for r in range(CPC):
        chunk = cid * CPC + r
        off = chunk * n if r == 0 else n
        add_off(dstoff, off)
        add_off(srcoff, off)
        pltpu.sync_copy(zeros_hbm, acc.at[pl.ds(sid * npt, npt), :])
        plsc.subcore_barrier()

        issue_gathers(0, bufs[0], chunk)
        issue_gathers(1, bufs[1], chunk)
        for b in (0, 1):
            bs = bufs[b]
            wait_gathers(b, bs, chunk)
            compute(b, bs)
            issue_out(b, bs, chunk)
            issue_gathers(b + 2, bs, chunk)

        @pl.loop(1, 61)
        def _(i):
            for p in range(2):
                b = 2 * i + p
                bs = bufs[p]
                wait_gathers(b, bs, chunk)
                wait_out(b - 2, bs, chunk)
                compute(b, bs)
                issue_out(b, bs, chunk)
                issue_gathers(b + 2, bs, chunk)

        for b in (122, 123, 124):
            bs = bufs[b % 2]
            wait_gathers(b, bs, chunk)
            wait_out(b - 2, bs, chunk)
            compute(b, bs)
            issue_out(b, bs, chunk)
            if b + 2 <= NBLK - 1:
                issue_gathers(b + 2, bs, chunk)
        for b in (123, 124):
            wait_out(b, bufs[b % 2], chunk)

        plsc.subcore_barrier()
        pltpu.sync_copy(acc.at[pl.ds(sid * npt, npt), :],
                        agg_hbm.at[chunk, pl.ds(sid * npt, npt), :])
        plsc.subcore_barrier()


def _sc_edge(dst2d, src2d, dtab, etab, btab, ctab, zeros, e, npad):
    mesh = plsc.VectorSubcoreMesh(core_axis_name="c", subcore_axis_name="s")
    kern = pl.kernel(
        _sc_edge_body,
        out_type=(
            jax.ShapeDtypeStruct((NCHUNK * e, C), jnp.float32),
            jax.ShapeDtypeStruct((NCHUNK, npad, 2 * C), jnp.float32),
        ),
        mesh=mesh,
        compiler_params=pltpu.CompilerParams(use_tc_tiling_on_sc=False),
        scratch_types=[
            pltpu.VMEM((NBLK, EB), jnp.int32),
            pltpu.VMEM((NBLK, EB), jnp.int32),
            pltpu.VMEM((NBLK, EB), jnp.int32),
            pltpu.VMEM((EB, C), jnp.float32),
            pltpu.VMEM((EB, C), jnp.float32),
            pltpu.VMEM((EB, C), jnp.float32),
            pltpu.VMEM((EB, C), jnp.float32),
            pltpu.VMEM((EB, C), jnp.float32),
            pltpu.VMEM((EB, 2 * C), jnp.float32),
            pltpu.VMEM((EB, C), jnp.float32),
            pltpu.VMEM((EB, C), jnp.float32),
            pltpu.VMEM((EB, C), jnp.float32),
            pltpu.VMEM((EB, C), jnp.float32),
            pltpu.VMEM((EB, C), jnp.float32),
            pltpu.VMEM((EB, 2 * C), jnp.float32),
            pltpu.VMEM_SHARED((npad, 2 * C), jnp.float32),
            pltpu.SemaphoreType.DMA,
            pltpu.SemaphoreType.DMA,
            pltpu.SemaphoreType.DMA,
            pltpu.SemaphoreType.DMA,
            pltpu.SemaphoreType.DMA,
            pltpu.SemaphoreType.DMA,
        ],
    )
    return kern(dst2d, src2d, dtab, etab, btab, ctab, zeros)


def _combine_body(x_ref, ax_ref, agg_ref, out_ref):
    for c in range(NCHUNK):
        s = pl.ds(c * C, C)
        num = agg_ref[c, :, :C]
        den = agg_ref[c, :, C:]
        t = (ax_ref[:, s] + num / (den + 1e-6)) * INV_SQRT
        out_ref[:, s] = x_ref[:, s] + jnp.maximum(t, 0.0)


def _combine(x, ax, agg):
    n, d = x.shape
    bn = 2000
    grid = (n // bn,)
    return pl.pallas_call(
        _combine_body,
        grid=grid,
        in_specs=[
            pl.BlockSpec((bn, d), lambda i: (i, 0)),
            pl.BlockSpec((bn, d), lambda i: (i, 0)),
            pl.BlockSpec((NCHUNK, bn, 2 * C), lambda i: (0, i, 0)),
        ],
        out_specs=pl.BlockSpec((bn, d), lambda i: (i, 0)),
        out_shape=jax.ShapeDtypeStruct((n, d), jnp.float32),
    )(x, ax, agg)


def _eout_body(ea_ref, ech_ref, out_ref):
    for c in range(NCHUNK):
        s = pl.ds(c * C, C)
        out_ref[:, s] = ea_ref[:, s] + jnp.maximum(ech_ref[c] * INV_SQRT, 0.0)


def _eout(ea, ech):
    e, d = ea.shape
    be = 2000
    grid = (e // be,)
    return pl.pallas_call(
        _eout_body,
        grid=grid,
        in_specs=[
            pl.BlockSpec((be, d), lambda i: (i, 0)),
            pl.BlockSpec((NCHUNK, be, C), lambda i: (0, i, 0)),
        ],
        out_specs=pl.BlockSpec((be, d), lambda i: (i, 0)),
        out_shape=jax.ShapeDtypeStruct((e, d), jnp.float32),
    )(ea, ech)


@jax.jit
def kernel(x, edge_index, edge_attr, W_A, b_A, W_B, b_B, W_C, b_C,
           W_D, b_D, W_E, b_E):
    n, d = x.shape
    e = edge_attr.shape[0]
    src = edge_index[0]
    dst = edge_index[1]

    wn = jnp.concatenate([W_A, W_B, W_D, W_E], axis=1)
    bn = jnp.concatenate([b_A, b_B, b_D, b_E]).reshape(1, 4 * d)
    ax, btab, dtab, etab = _proj_nodes(x, wn, bn)
    ctab = _proj_edges(edge_attr, W_C, b_C.reshape(1, d))

    npad = 10240
    zeros = jnp.zeros((npad // 16, 2 * C), jnp.float32)
    ech, agg = _sc_edge(
        dst.reshape(16, NBLK, EB),
        src.reshape(16, NBLK, EB),
        dtab.reshape(NCHUNK * n, C),
        etab.reshape(NCHUNK * n, C),
        btab.reshape(NCHUNK * n, C),
        ctab.reshape(NCHUNK * e, C),
        zeros, e, npad)

    xout = _combine(x, ax, agg)
    eout = _eout(edge_attr, ech.reshape(NCHUNK, e, C))
    return (xout, eout)


# speedup vs baseline: 3.6518x; 2.2021x over previous
import functools

import jax
import jax.numpy as jnp
from jax import lax
from jax.experimental import pallas as pl
from jax.experimental.pallas import tpu as pltpu
from jax.experimental.pallas import tpu_sc as plsc

C = 32
NCHUNK = 8
CPC = NCHUNK // 2
EB = 80
NBLK = 125
INV_SQRT = 1.0 / (1.0 + 1e-5) ** 0.5


def _proj_nodes_body(x_ref, w_ref, b_ref, ax_ref, bt_ref, dt_ref, et_ref):
    y = jnp.dot(x_ref[...], w_ref[...], preferred_element_type=jnp.float32)
    y = y + b_ref[...]
    d = ax_ref.shape[1]
    ax_ref[...] = y[:, :d]
    for c in range(NCHUNK):
        bt_ref[c] = y[:, d + c * C:d + (c + 1) * C]
        dt_ref[c] = y[:, 2 * d + c * C:2 * d + (c + 1) * C]
        et_ref[c] = y[:, 3 * d + c * C:3 * d + (c + 1) * C]


def _proj_nodes(x, w, b):
    n, d = x.shape
    bn = 1000
    grid = (n // bn,)
    return pl.pallas_call(
        _proj_nodes_body,
        grid=grid,
        in_specs=[
            pl.BlockSpec((bn, d), lambda i: (i, 0)),
            pl.BlockSpec((d, 4 * d), lambda i: (0, 0)),
            pl.BlockSpec((1, 4 * d), lambda i: (0, 0)),
        ],
        out_specs=[
            pl.BlockSpec((bn, d), lambda i: (i, 0)),
            pl.BlockSpec((NCHUNK, bn, C), lambda i: (0, i, 0)),
            pl.BlockSpec((NCHUNK, bn, C), lambda i: (0, i, 0)),
            pl.BlockSpec((NCHUNK, bn, C), lambda i: (0, i, 0)),
        ],
        out_shape=[
            jax.ShapeDtypeStruct((n, d), jnp.float32),
            jax.ShapeDtypeStruct((NCHUNK, n, C), jnp.float32),
            jax.ShapeDtypeStruct((NCHUNK, n, C), jnp.float32),
            jax.ShapeDtypeStruct((NCHUNK, n, C), jnp.float32),
        ],
    )(x, w, b)


def _proj_edges_body(ea_ref, w_ref, b_ref, ct_ref):
    y = jnp.dot(ea_ref[...], w_ref[...], preferred_element_type=jnp.float32)
    y = y + b_ref[...]
    h = ct_ref.shape[2]
    ct_ref[0] = y[:, :h]
    ct_ref[1] = y[:, h:]


def _proj_edges(ea, w, b):
    e, d = ea.shape
    be = 2000
    grid = (e // be,)
    return pl.pallas_call(
        _proj_edges_body,
        grid=grid,
        in_specs=[
            pl.BlockSpec((be, d), lambda i: (i, 0)),
            pl.BlockSpec((d, d), lambda i: (0, 0)),
            pl.BlockSpec((1, d), lambda i: (0, 0)),
        ],
        out_specs=[pl.BlockSpec((2, be, d // 2), lambda i: (0, i, 0))],
        out_shape=[jax.ShapeDtypeStruct((2, e, d // 2), jnp.float32)],
    )(ea, w, b)[0]


def _sc_edge_body(dst2d_hbm, src2d_hbm, dtab, etab, btab, ctab,
                  zeros_hbm, ech_hbm, agg_hbm,
                  dst2d, dstoff, srcoff,
                  dx0, ex0, bx0, ce0, eo0, ms0,
                  dx1, ex1, bx1, ce1, eo1, ms1,
                  acc, sg0, sg1, ss0, ss1, se0, se1):
    n = dtab.shape[0] // NCHUNK
    npad = agg_hbm.shape[1]
    cid = lax.axis_index("c")
    sid = lax.axis_index("s")
    ept = NBLK * EB
    npt = npad // 16
    tbase = sid * ept

    pltpu.sync_copy(dst2d_hbm.at[sid], dst2d)
    pltpu.sync_copy(dst2d_hbm.at[sid], dstoff)
    pltpu.sync_copy(src2d_hbm.at[sid], srcoff)

    bufs = ((dx0, ex0, bx0, ce0, eo0, ms0, sg0, ss0, se0),
            (dx1, ex1, bx1, ce1, eo1, ms1, sg1, ss1, se1))

    def add_off(ref, off):
        vec = jnp.full((16,), off, dtype=jnp.int32)

        @pl.loop(0, NBLK)
        def _(i):
            for j in range(EB // 16):
                sl = pl.ds(j * 16, 16)
                ref[i, sl] = ref[i, sl] + vec

    def issue_gathers(b, bs, r):
        dx, ex, bx, ce, _, _, sg, _, _ = bs
        pltpu.async_copy(dtab.at[dstoff.at[b]], dx, sg)
        pltpu.async_copy(etab.at[srcoff.at[b]], ex, sg)
        pltpu.async_copy(btab.at[srcoff.at[b]], bx, sg)
        pltpu.async_copy(
            ctab.at[cid, pl.ds(tbase + b * EB, EB), pl.ds(r * C, C)], ce, sg)

    def wait_gathers(b, bs, r):
        dx, ex, bx, ce, _, _, sg, _, _ = bs
        pltpu.make_async_copy(dtab.at[dstoff.at[b]], dx, sg).wait()
        pltpu.make_async_copy(etab.at[srcoff.at[b]], ex, sg).wait()
        pltpu.make_async_copy(btab.at[srcoff.at[b]], bx, sg).wait()
        pltpu.make_async_copy(
            ctab.at[cid, pl.ds(tbase + b * EB, EB), pl.ds(r * C, C)],
            ce, sg).wait()

    def compute(b, bs):
        dx, ex, bx, ce, eo, ms, _, _, _ = bs

        @plsc.parallel_loop(0, EB)
        def _(row):
            for j in range(C // 16):
                s = pl.ds(j * 16, 16)
                ev = dx[row, s] + ex[row, s] + ce[row, s]
                eo[row, s] = ev
                sig = 1.0 / (1.0 + jnp.exp(-ev))
                ms[row, s] = sig * bx[row, s]
                ms[row, pl.ds(C + j * 16, 16)] = sig

    def issue_out(b, bs, r):
        _, _, _, _, eo, ms, _, ss, se = bs
        pltpu.async_copy(
            eo, ech_hbm.at[cid, pl.ds(tbase + b * EB, EB), pl.ds(r * C, C)],
            se)
        pltpu.async_copy(ms, acc.at[dst2d.at[b]], ss, add=True)

    def wait_out(b, bs, r):
        _, _, _, _, eo, ms, _, ss, se = bs
        pltpu.make_async_copy(
            eo, ech_hbm.at[cid, pl.ds(tbase + b * EB, EB), pl.ds(r * C, C)],
            se).wait()
        pltpu.make_async_copy(ms, acc.at[dst2d.at[b]], ss).wait()

    for r in range(CPC):
        chunk = cid * CPC + r
        off = chunk * n if r == 0 else n
        add_off(dstoff, off)
        add_off(srcoff, off)
        pltpu.sync_copy(zeros_hbm, acc.at[pl.ds(sid * npt, npt), :])
        plsc.subcore_barrier()

        issue_gathers(0, bufs[0], r)
        issue_gathers(1, bufs[1], r)
        for b in (0, 1):
            bs = bufs[b]
            wait_gathers(b, bs, r)
            compute(b, bs)
            issue_out(b, bs, r)
            issue_gathers(b + 2, bs, r)

        @pl.loop(1, 61)
        def _(i):
            for p in range(2):
                b = 2 * i + p
                bs = bufs[p]
                wait_gathers(b, bs, r)
                wait_out(b - 2, bs, r)
                compute(b, bs)
                issue_out(b, bs, r)
                issue_gathers(b + 2, bs, r)

        for b in (122, 123, 124):
            bs = bufs[b % 2]
            wait_gathers(b, bs, r)
            wait_out(b - 2, bs, r)
            compute(b, bs)
            issue_out(b, bs, r)
            if b + 2 <= NBLK - 1:
                issue_gathers(b + 2, bs, r)
        for b in (123, 124):
            wait_out(b, bufs[b % 2], r)

        plsc.subcore_barrier()
        pltpu.sync_copy(
            acc.at[pl.ds(sid * npt, npt), :],
            agg_hbm.at[cid * CPC // 2 + r // 2, pl.ds(sid * npt, npt),
                       pl.ds((r % 2) * 2 * C, 2 * C)])
        plsc.subcore_barrier()


def _sc_edge(dst2d, src2d, dtab, etab, btab, ctab, zeros, e, npad):
    mesh = plsc.VectorSubcoreMesh(core_axis_name="c", subcore_axis_name="s")
    kern = pl.kernel(
        _sc_edge_body,
        out_type=(
            jax.ShapeDtypeStruct((2, e, 4 * C), jnp.float32),
            jax.ShapeDtypeStruct((NCHUNK // 2, npad, 4 * C), jnp.float32),
        ),
        mesh=mesh,
        compiler_params=pltpu.CompilerParams(use_tc_tiling_on_sc=False),
        scratch_types=[
            pltpu.VMEM((NBLK, EB), jnp.int32),
            pltpu.VMEM((NBLK, EB), jnp.int32),
            pltpu.VMEM((NBLK, EB), jnp.int32),
            pltpu.VMEM((EB, C), jnp.float32),
            pltpu.VMEM((EB, C), jnp.float32),
            pltpu.VMEM((EB, C), jnp.float32),
            pltpu.VMEM((EB, C), jnp.float32),
            pltpu.VMEM((EB, C), jnp.float32),
            pltpu.VMEM((EB, 2 * C), jnp.float32),
            pltpu.VMEM((EB, C), jnp.float32),
            pltpu.VMEM((EB, C), jnp.float32),
            pltpu.VMEM((EB, C), jnp.float32),
            pltpu.VMEM((EB, C), jnp.float32),
            pltpu.VMEM((EB, C), jnp.float32),
            pltpu.VMEM((EB, 2 * C), jnp.float32),
            pltpu.VMEM_SHARED((npad, 2 * C), jnp.float32),
            pltpu.SemaphoreType.DMA,
            pltpu.SemaphoreType.DMA,
            pltpu.SemaphoreType.DMA,
            pltpu.SemaphoreType.DMA,
            pltpu.SemaphoreType.DMA,
            pltpu.SemaphoreType.DMA,
        ],
    )
    return kern(dst2d, src2d, dtab, etab, btab, ctab, zeros)


def _combine_body(x_ref, ax_ref, agg_ref, out_ref):
    for c in range(NCHUNK):
        s = pl.ds(c * C, C)
        base = (c % 2) * 2 * C
        num = agg_ref[c // 2, :, base:base + C]
        den = agg_ref[c // 2, :, base + C:base + 2 * C]
        t = (ax_ref[:, s] + num / (den + 1e-6)) * INV_SQRT
        out_ref[:, s] = x_ref[:, s] + jnp.maximum(t, 0.0)


def _combine(x, ax, agg):
    n, d = x.shape
    bn = 2000
    grid = (n // bn,)
    return pl.pallas_call(
        _combine_body,
        grid=grid,
        in_specs=[
            pl.BlockSpec((bn, d), lambda i: (i, 0)),
            pl.BlockSpec((bn, d), lambda i: (i, 0)),
            pl.BlockSpec((NCHUNK // 2, bn, 4 * C), lambda i: (0, i, 0)),
        ],
        out_specs=pl.BlockSpec((bn, d), lambda i: (i, 0)),
        out_shape=jax.ShapeDtypeStruct((n, d), jnp.float32),
    )(x, ax, agg)


def _eout_body(ea_ref, ech_ref, out_ref):
    for c in range(NCHUNK):
        s = pl.ds(c * C, C)
        half = ech_ref[c // 4, :, (c % 4) * C:(c % 4 + 1) * C]
        out_ref[:, s] = ea_ref[:, s] + jnp.maximum(half * INV_SQRT, 0.0)


def _eout(ea, ech):
    e, d = ea.shape
    be = 2000
    grid = (e // be,)
    return pl.pallas_call(
        _eout_body,
        grid=grid,
        in_specs=[
            pl.BlockSpec((be, d), lambda i: (i, 0)),
            pl.BlockSpec((2, be, 4 * C), lambda i: (0, i, 0)),
        ],
        out_specs=pl.BlockSpec((be, d), lambda i: (i, 0)),
        out_shape=jax.ShapeDtypeStruct((e, d), jnp.float32),
    )(ea, ech)


@jax.jit
def kernel(x, edge_index, edge_attr, W_A, b_A, W_B, b_B, W_C, b_C,
           W_D, b_D, W_E, b_E):
    n, d = x.shape
    e = edge_attr.shape[0]
    src = edge_index[0]
    dst = edge_index[1]

    wn = jnp.concatenate([W_A, W_B, W_D, W_E], axis=1)
    bn = jnp.concatenate([b_A, b_B, b_D, b_E]).reshape(1, 4 * d)
    ax, btab, dtab, etab = _proj_nodes(x, wn, bn)
    ctab = _proj_edges(edge_attr, W_C, b_C.reshape(1, d))

    npad = 10240
    zeros = jnp.zeros((npad // 16, 2 * C), jnp.float32)
    ech, agg = _sc_edge(
        dst.reshape(16, NBLK, EB),
        src.reshape(16, NBLK, EB),
        dtab.reshape(NCHUNK * n, C),
        etab.reshape(NCHUNK * n, C),
        btab.reshape(NCHUNK * n, C),
        ctab,
        zeros, e, npad)

    xout = _combine(x, ax, agg)
    eout = _eout(edge_attr, ech)
    return (xout, eout)


# speedup vs baseline: 3.7711x; 1.0327x over previous
import functools

import jax
import jax.numpy as jnp
from jax import lax
from jax.experimental import pallas as pl
from jax.experimental.pallas import tpu as pltpu
from jax.experimental.pallas import tpu_sc as plsc

C = 32
NCHUNK = 8
CPC = NCHUNK // 2
EB = 80
NBLK = 125
INV_SQRT = 1.0 / (1.0 + 1e-5) ** 0.5


def _proj_nodes_body(x_ref, w_ref, b_ref, ax_ref, dt_ref, eb_ref):
    y = jnp.dot(x_ref[...], w_ref[...], preferred_element_type=jnp.float32)
    y = y + b_ref[...]
    d = ax_ref.shape[1]
    ax_ref[...] = y[:, :d]
    for k in range(NCHUNK):
        dt_ref[k] = y[:, 2 * d + k * C:2 * d + (k + 1) * C]
        eb_ref[k, :, 0:C] = y[:, 3 * d + k * C:3 * d + (k + 1) * C]
        eb_ref[k, :, C:2 * C] = y[:, d + k * C:d + (k + 1) * C]


def _proj_nodes(x, w, b):
    n, d = x.shape
    bn = 1000
    grid = (n // bn,)
    return pl.pallas_call(
        _proj_nodes_body,
        grid=grid,
        in_specs=[
            pl.BlockSpec((bn, d), lambda i: (i, 0)),
            pl.BlockSpec((d, 4 * d), lambda i: (0, 0)),
            pl.BlockSpec((1, 4 * d), lambda i: (0, 0)),
        ],
        out_specs=[
            pl.BlockSpec((bn, d), lambda i: (i, 0)),
            pl.BlockSpec((NCHUNK, bn, C), lambda i: (0, i, 0)),
            pl.BlockSpec((NCHUNK, bn, 2 * C), lambda i: (0, i, 0)),
        ],
        out_shape=[
            jax.ShapeDtypeStruct((n, d), jnp.float32),
            jax.ShapeDtypeStruct((NCHUNK, n, C), jnp.float32),
            jax.ShapeDtypeStruct((NCHUNK, n, 2 * C), jnp.float32),
        ],
    )(x, w, b)


def _proj_edges_body(ea_ref, w_ref, b_ref, ct_ref):
    y = jnp.dot(ea_ref[...], w_ref[...], preferred_element_type=jnp.float32)
    y = y + b_ref[...]
    h = ct_ref.shape[2]
    ct_ref[0] = y[:, :h]
    ct_ref[1] = y[:, h:]


def _proj_edges(ea, w, b):
    e, d = ea.shape
    be = 2000
    grid = (e // be,)
    return pl.pallas_call(
        _proj_edges_body,
        grid=grid,
        in_specs=[
            pl.BlockSpec((be, d), lambda i: (i, 0)),
            pl.BlockSpec((d, d), lambda i: (0, 0)),
            pl.BlockSpec((1, d), lambda i: (0, 0)),
        ],
        out_specs=[pl.BlockSpec((2, be, d // 2), lambda i: (0, i, 0))],
        out_shape=[jax.ShapeDtypeStruct((2, e, d // 2), jnp.float32)],
    )(ea, w, b)[0]


def _sc_edge_body(dst2d_hbm, src2d_hbm, dtab, ebtab, ctab,
                  zeros_hbm, ech_hbm, agg_hbm,
                  dst2d, src2d, bdst, bsrc,
                  dx0, eb0, ce0, eo0, ms0,
                  dx1, eb1, ce1, eo1, ms1,
                  acc, sg0, sg1, ss0, ss1, se0, se1):
    npad = agg_hbm.shape[1]
    nn = dtab.shape[0] // NCHUNK
    cid = lax.axis_index("c")
    sid = lax.axis_index("s")
    ept = NBLK * EB
    npt = npad // 16
    tbase = sid * ept

    pltpu.sync_copy(dst2d_hbm.at[sid], dst2d)
    pltpu.sync_copy(src2d_hbm.at[sid], src2d)

    bufs = ((dx0, eb0, ce0, eo0, ms0, sg0, ss0, se0),
            (dx1, eb1, ce1, eo1, ms1, sg1, ss1, se1))

    def issue_gathers(b, bs, r):
        dx, eb, ce, _, _, sg, _, _ = bs
        pltpu.async_copy(dtab.at[bdst.at[b]], dx, sg)
        pltpu.async_copy(ebtab.at[bsrc.at[b]], eb, sg)
        pltpu.async_copy(
            ctab.at[cid, pl.ds(tbase + b * EB, EB), pl.ds(r * C, C)], ce, sg)

    def wait_gathers(b, bs, r):
        dx, eb, ce, _, _, sg, _, _ = bs
        pltpu.make_async_copy(dtab.at[bdst.at[b]], dx, sg).wait()
        pltpu.make_async_copy(ebtab.at[bsrc.at[b]], eb, sg).wait()
        pltpu.make_async_copy(
            ctab.at[cid, pl.ds(tbase + b * EB, EB), pl.ds(r * C, C)],
            ce, sg).wait()

    def compute(b, bs):
        dx, eb, ce, eo, ms, _, _, _ = bs

        @plsc.parallel_loop(0, EB)
        def _(row):
            for j in range(C // 16):
                s = pl.ds(j * 16, 16)
                ev = dx[row, s] + eb[row, s] + ce[row, s]
                eo[row, s] = ev
                sig = 1.0 / (1.0 + jnp.exp(-ev))
                ms[row, s] = sig * eb[row, pl.ds(C + j * 16, 16)]
                ms[row, pl.ds(C + j * 16, 16)] = sig

    def issue_out(b, bs, r):
        _, _, _, eo, ms, _, ss, se = bs
        pltpu.async_copy(
            eo, ech_hbm.at[cid, pl.ds(tbase + b * EB, EB), pl.ds(r * C, C)],
            se)
        pltpu.async_copy(ms, acc.at[dst2d.at[b]], ss, add=True)

    def wait_out(b, bs, r):
        _, _, _, eo, ms, _, ss, se = bs
        pltpu.make_async_copy(
            eo, ech_hbm.at[cid, pl.ds(tbase + b * EB, EB), pl.ds(r * C, C)],
            se).wait()
        pltpu.make_async_copy(ms, acc.at[dst2d.at[b]], ss).wait()

    for r in range(CPC):
        bias = (cid * CPC + r) * nn

        @plsc.parallel_loop(0, NBLK)
        def _(i):
            for j in range(EB // 16):
                s = pl.ds(j * 16, 16)
                bdst[i, s] = dst2d[i, s] + bias
                bsrc[i, s] = src2d[i, s] + bias

        pltpu.sync_copy(zeros_hbm, acc.at[pl.ds(sid * npt, npt), :])
        plsc.subcore_barrier()

        issue_gathers(0, bufs[0], r)
        issue_gathers(1, bufs[1], r)
        for b in (0, 1):
            bs = bufs[b]
            wait_gathers(b, bs, r)
            compute(b, bs)
            issue_out(b, bs, r)
            issue_gathers(b + 2, bs, r)

        @pl.loop(1, 61)
        def _(i):
            for p in range(2):
                b = 2 * i + p
                bs = bufs[p]
                wait_gathers(b, bs, r)
                wait_out(b - 2, bs, r)
                compute(b, bs)
                issue_out(b, bs, r)
                issue_gathers(b + 2, bs, r)

        for b in (122, 123, 124):
            bs = bufs[b % 2]
            wait_gathers(b, bs, r)
            wait_out(b - 2, bs, r)
            compute(b, bs)
            issue_out(b, bs, r)
            if b + 2 <= NBLK - 1:
                issue_gathers(b + 2, bs, r)
        for b in (123, 124):
            wait_out(b, bufs[b % 2], r)

        plsc.subcore_barrier()
        pltpu.sync_copy(
            acc.at[pl.ds(sid * npt, npt), :],
            agg_hbm.at[cid * CPC // 2 + r // 2, pl.ds(sid * npt, npt),
                       pl.ds((r % 2) * 2 * C, 2 * C)])
        plsc.subcore_barrier()


def _sc_edge(dst2d, src2d, dtab, ebtab, ctab, zeros, e, npad):
    mesh = plsc.VectorSubcoreMesh(core_axis_name="c", subcore_axis_name="s")
    kern = pl.kernel(
        _sc_edge_body,
        out_type=(
            jax.ShapeDtypeStruct((2, e, 4 * C), jnp.float32),
            jax.ShapeDtypeStruct((NCHUNK // 2, npad, 4 * C), jnp.float32),
        ),
        mesh=mesh,
        compiler_params=pltpu.CompilerParams(use_tc_tiling_on_sc=False),
        scratch_types=[
            pltpu.VMEM((NBLK, EB), jnp.int32),
            pltpu.VMEM((NBLK, EB), jnp.int32),
            pltpu.VMEM((NBLK, EB), jnp.int32),
            pltpu.VMEM((NBLK, EB), jnp.int32),
            pltpu.VMEM((EB, C), jnp.float32),
            pltpu.VMEM((EB, 2 * C), jnp.float32),
            pltpu.VMEM((EB, C), jnp.float32),
            pltpu.VMEM((EB, C), jnp.float32),
            pltpu.VMEM((EB, 2 * C), jnp.float32),
            pltpu.VMEM((EB, C), jnp.float32),
            pltpu.VMEM((EB, 2 * C), jnp.float32),
            pltpu.VMEM((EB, C), jnp.float32),
            pltpu.VMEM((EB, C), jnp.float32),
            pltpu.VMEM((EB, 2 * C), jnp.float32),
            pltpu.VMEM_SHARED((npad, 2 * C), jnp.float32),
            pltpu.SemaphoreType.DMA,
            pltpu.SemaphoreType.DMA,
            pltpu.SemaphoreType.DMA,
            pltpu.SemaphoreType.DMA,
            pltpu.SemaphoreType.DMA,
            pltpu.SemaphoreType.DMA,
        ],
    )
    return kern(dst2d, src2d, dtab, ebtab, ctab, zeros)


def _combine_body(x_ref, ax_ref, agg_ref, out_ref):
    for c in range(NCHUNK):
        s = pl.ds(c * C, C)
        base = (c % 2) * 2 * C
        num = agg_ref[c // 2, :, base:base + C]
        den = agg_ref[c // 2, :, base + C:base + 2 * C]
        t = (ax_ref[:, s] + num / (den + 1e-6)) * INV_SQRT
        out_ref[:, s] = x_ref[:, s] + jnp.maximum(t, 0.0)


def _combine(x, ax, agg):
    n, d = x.shape
    bn = 2000
    grid = (n // bn,)
    return pl.pallas_call(
        _combine_body,
        grid=grid,
        in_specs=[
            pl.BlockSpec((bn, d), lambda i: (i, 0)),
            pl.BlockSpec((bn, d), lambda i: (i, 0)),
            pl.BlockSpec((NCHUNK // 2, bn, 4 * C), lambda i: (0, i, 0)),
        ],
        out_specs=pl.BlockSpec((bn, d), lambda i: (i, 0)),
        out_shape=jax.ShapeDtypeStruct((n, d), jnp.float32),
    )(x, ax, agg)


def _eout_body(ea_ref, ech_ref, out_ref):
    for c in range(NCHUNK):
        s = pl.ds(c * C, C)
        half = ech_ref[c // 4, :, (c % 4) * C:(c % 4 + 1) * C]
        out_ref[:, s] = ea_ref[:, s] + jnp.maximum(half * INV_SQRT, 0.0)


def _eout(ea, ech):
    e, d = ea.shape
    be = 2000
    grid = (e // be,)
    return pl.pallas_call(
        _eout_body,
        grid=grid,
        in_specs=[
            pl.BlockSpec((be, d), lambda i: (i, 0)),
            pl.BlockSpec((2, be, 4 * C), lambda i: (0, i, 0)),
        ],
        out_specs=pl.BlockSpec((be, d), lambda i: (i, 0)),
        out_shape=jax.ShapeDtypeStruct((e, d), jnp.float32),
    )(ea, ech)


@jax.jit
def kernel(x, edge_index, edge_attr, W_A, b_A, W_B, b_B, W_C, b_C,
           W_D, b_D, W_E, b_E):
    n, d = x.shape
    e = edge_attr.shape[0]
    src = edge_index[0]
    dst = edge_index[1]

    wn = jnp.concatenate([W_A, W_B, W_D, W_E], axis=1)
    bn = jnp.concatenate([b_A, b_B, b_D, b_E]).reshape(1, 4 * d)
    ax, dtab, ebtab = _proj_nodes(x, wn, bn)
    ctab = _proj_edges(edge_attr, W_C, b_C.reshape(1, d))

    npad = 10240
    zeros = jnp.zeros((npad // 16, 2 * C), jnp.float32)
    ech, agg = _sc_edge(
        dst.reshape(16, NBLK, EB),
        src.reshape(16, NBLK, EB),
        dtab.reshape(NCHUNK * n, C),
        ebtab.reshape(NCHUNK * n, 2 * C),
        ctab,
        zeros, e, npad)

    xout = _combine(x, ax, agg)
    eout = _eout(edge_attr, ech)
    return (xout, eout)
